# baseline (speedup 1.0000x reference)
import functools

import jax
import jax.numpy as jnp
from jax import lax
from jax.experimental import pallas as pl
from jax.experimental.pallas import tpu as pltpu
from jax.experimental.pallas import tpu_sc as plsc

N_NODES = 10000
N_EDGES = 320000
EPAD = 327680
NPAD = EPAD - N_EDGES
C_U = 32
HID = 128

NC = 2
NS = 16
NW = NC * NS
NSLAB = 4
ESLAB = EPAD // NSLAB
EPW = ESLAB // NW
GCHUNK = 640
NGCHUNK = EPW // GCHUNK
NJOB = 2 * NGCHUNK
CHUNK = 640
NCHUNK = EPW // CHUNK
ACC_W = 48
PW = HID // 2

_mesh = plsc.VectorSubcoreMesh(core_axis_name="c", subcore_axis_name="s")


def _pre_body(ut_ref, x0t_ref, x1t_ref, w0_ref, b0_ref, pt_ref, qt_ref,
              utp_ref):
    w0 = w0_ref[...]
    a = w0[:, 0:32]
    b = w0[:, 32:64]
    c = w0[:, 64:96]
    d = w0[:, 96:128]
    dn = (((1,), (1,)), ((), ()))
    pt = lax.dot_general(ut_ref[...], a, dn)
    pt = pt + lax.dot_general(x0t_ref[...], c, dn)
    pt = pt + lax.dot_general(x1t_ref[...], d, dn)
    pt = pt + b0_ref[...]
    qt = lax.dot_general(ut_ref[...], b, dn)

    def pack(x):
        xb = x.astype(jnp.bfloat16)
        lo = lax.convert_element_type(
            lax.bitcast_convert_type(xb[:, 0:64], jnp.uint16), jnp.uint32)
        hi = lax.convert_element_type(
            lax.bitcast_convert_type(xb[:, 64:128], jnp.uint16), jnp.uint32)
        return lax.bitcast_convert_type(lo | (hi << 16), jnp.int32)

    pt_ref[...] = pack(pt)
    qt_ref[...] = pack(qt)
    ub = ut_ref[...].astype(jnp.bfloat16)
    ulo = lax.convert_element_type(
        lax.bitcast_convert_type(ub[:, 0:16], jnp.uint16), jnp.uint32)
    uhi = lax.convert_element_type(
        lax.bitcast_convert_type(ub[:, 16:32], jnp.uint16), jnp.uint32)
    utp_ref[...] = lax.bitcast_convert_type(ulo | (uhi << 16), jnp.int32)


def _precompute(ut, x0t, x1t, w0, b0row):
    return pl.pallas_call(
        _pre_body,
        out_shape=(
            jax.ShapeDtypeStruct((N_NODES, HID // 2), jnp.int32),
            jax.ShapeDtypeStruct((N_NODES, HID // 2), jnp.int32),
            jax.ShapeDtypeStruct((N_NODES, C_U // 2), jnp.int32),
        ),
    )(ut, x0t, x1t, w0, b0row)


def _gather_body(pt_hbm, qt_hbm, i0_hbm, i1_hbm, g0_hbm, g1_hbm,
                 idx0_v, idx1_v, rows_v, si0, si1, sg0, sg1, sw0, sw1):
    wid = lax.axis_index("s") * NC + lax.axis_index("c")
    tile_base = wid * EPW
    sem_i = (si0, si1)
    sem_g = (sg0, sg1)
    sem_w = (sw0, sw1)
    isrc = (i0_hbm, i1_hbm)
    tbl = (pt_hbm, qt_hbm)
    dst = (g0_hbm, g1_hbm)
    idxb = (idx0_v, idx1_v)

    def idx_cp(j, b):
        base = tile_base + (j >> 1) * GCHUNK
        return pltpu.make_async_copy(
            isrc[b].at[pl.ds(base, GCHUNK)], idxb[b], sem_i[b])

    def gat_cp(b):
        return pltpu.make_async_copy(
            tbl[b].at[idxb[b]], rows_v.at[b], sem_g[b])

    def wb_cp(j, b):
        base = tile_base + (j >> 1) * GCHUNK
        return pltpu.make_async_copy(
            rows_v.at[b], dst[b].at[pl.ds(base, GCHUNK)], sem_w[b])

    idx_cp(0, 0).start()
    idx_cp(1, 1).start()
    idx_cp(0, 0).wait()
    gat_cp(0).start()
    gat_cp(0).wait()
    wb_cp(0, 0).start()
    idx_cp(2, 0).start()
    idx_cp(1, 1).wait()
    gat_cp(1).start()
    gat_cp(1).wait()
    wb_cp(1, 1).start()
    idx_cp(3, 1).start()
    wb_cp(0, 0).wait()
    idx_cp(2, 0).wait()
    gat_cp(0).start()

    def steady(j, b):
        gat_cp(b).wait()
        wb_cp(j, b).start()
        idx_cp(j + 2, b).start()
        wb_cp(j - 1, 1 - b).wait()
        idx_cp(j + 1, 1 - b).wait()
        gat_cp(1 - b).start()

    @pl.loop(2, NJOB - 2, step=2)
    def _(jv):
        steady(jv, 0)
        steady(jv + 1, 1)

    j = NJOB - 2
    gat_cp(0).wait()
    wb_cp(j, 0).start()
    wb_cp(j - 1, 1).wait()
    idx_cp(j + 1, 1).wait()
    gat_cp(1).start()
    gat_cp(1).wait()
    wb_cp(j + 1, 1).start()
    wb_cp(j, 0).wait()
    wb_cp(j + 1, 1).wait()


_gather = functools.partial(
    pl.kernel,
    out_type=(
        jax.ShapeDtypeStruct((ESLAB, PW), jnp.int32),
        jax.ShapeDtypeStruct((ESLAB, PW), jnp.int32),
    ),
    mesh=_mesh,
    compiler_params=pltpu.CompilerParams(use_tc_tiling_on_sc=False),
    scratch_types=[
        pltpu.VMEM((GCHUNK,), jnp.int32),
        pltpu.VMEM((GCHUNK,), jnp.int32),
        pltpu.VMEM((2, GCHUNK, PW), jnp.int32),
        pltpu.SemaphoreType.DMA,
        pltpu.SemaphoreType.DMA,
        pltpu.SemaphoreType.DMA,
        pltpu.SemaphoreType.DMA,
        pltpu.SemaphoreType.DMA,
        pltpu.SemaphoreType.DMA,
    ],
)(_gather_body)


BE = 2048
NBLK = ESLAB // BE


def _mlp_body(slab_off, g0_ref, g1_ref, w1_ref, b1_ref, w2_ref, b2_ref,
              w3_ref, b3_ref, w4_ref, b4_ref, s_ref):
    dn = (((1,), (1,)), ((), ()))
    f32 = jnp.float32

    def unpack(gref):
        gu = lax.bitcast_convert_type(gref[...], jnp.uint32)
        lo = lax.bitcast_convert_type(gu << 16, f32)
        hi = lax.bitcast_convert_type(gu & jnp.uint32(0xFFFF0000), f32)
        return jnp.concatenate([lo, hi], axis=1)

    h = unpack(g0_ref) + unpack(g1_ref)
    for w_ref, b_ref in ((w1_ref, b1_ref), (w2_ref, b2_ref), (w3_ref, b3_ref)):
        h = jax.nn.gelu(h).astype(jnp.bfloat16)
        h = lax.dot_general(h, w_ref[...], dn,
                            preferred_element_type=f32) + b_ref[...]
    h = jax.nn.gelu(h).astype(jnp.bfloat16)
    s = lax.dot_general(w4_ref[...].astype(jnp.bfloat16), h, dn,
                        preferred_element_type=f32) + b4_ref[0, 0]
    i = pl.program_id(0)
    eid = slab_off + i * BE + lax.broadcasted_iota(jnp.int32, (1, BE), 1)
    s = jnp.where(eid < N_EDGES, s, 0.0)
    s_ref[...] = s.reshape(BE)


def _mlp(slab_off, g0, g1, w1, b1r, w2, b2r, w3, b3r, w4, b4):
    wspec = pl.BlockSpec((HID, HID), lambda i: (0, 0))
    bspec = pl.BlockSpec((1, HID), lambda i: (0, 0))
    return pl.pallas_call(
        functools.partial(_mlp_body, slab_off),
        grid=(NBLK,),
        in_specs=[
            pl.BlockSpec((BE, PW), lambda i: (i, 0)),
            pl.BlockSpec((BE, PW), lambda i: (i, 0)),
            wspec, bspec, wspec, bspec, wspec, bspec,
            pl.BlockSpec((1, HID), lambda i: (0, 0)),
            pl.BlockSpec((1, 1), lambda i: (0, 0)),
        ],
        out_specs=pl.BlockSpec((BE,), lambda i: (i,)),
        out_shape=jax.ShapeDtypeStruct((ESLAB,), jnp.float32),
    )(g0, g1, w1, b1r, w2, b2r, w3, b3r, w4, b4)


def _scatter_body(utp_hbm, i0_hbm, i1_hbm, s_hbm, zero_hbm, out_hbm,
                  idx0_v, idx1_v, s_v, urows_v, scaled_v, acc_sh,
                  sl0, sl1, sl2, sl3, sg0, sg1, ss0, ss1):
    cid = lax.axis_index("c")
    sid = lax.axis_index("s")
    wid = sid * NC + cid
    sem_l = (sl0, sl1, sl2, sl3)
    sem_g = (sg0, sg1)
    sem_s = (ss0, ss1)

    @pl.when(sid == 0)
    def _():
        pltpu.sync_copy(zero_hbm, acc_sh)

    cnt_vec = jnp.where(lax.iota(jnp.int32, 16) == 0,
                        jnp.float32(1.0), jnp.float32(0.0))

    def init_body(e, carry):
        scaled_v[0, e, 32:48] = cnt_vec
        scaled_v[1, e, 32:48] = cnt_vec
        return carry

    lax.fori_loop(0, CHUNK, init_body, 0)
    plsc.subcore_barrier()

    def ld_cps(c):
        b = c % 4
        base = wid * EPW + c * CHUNK
        return (
            pltpu.make_async_copy(
                i0_hbm.at[pl.ds(base, CHUNK)], idx0_v.at[b], sem_l[b]),
            pltpu.make_async_copy(
                i1_hbm.at[pl.ds(base, CHUNK)], idx1_v.at[b], sem_l[b]),
            pltpu.make_async_copy(
                s_hbm.at[pl.ds(base, CHUNK)], s_v.at[b], sem_l[b]),
        )

    def gat_cp(c):
        return pltpu.make_async_copy(
            utp_hbm.at[idx0_v.at[c % 4]], urows_v.at[c % 2], sem_g[c % 2])

    def sca_start(c):
        pltpu.async_copy(
            scaled_v.at[c % 2], acc_sh.at[idx1_v.at[c % 4]], sem_s[c % 2],
            add=True)

    def sca_wait(c):
        pltpu.make_async_copy(
            scaled_v.at[c % 2], acc_sh.at[idx1_v.at[c % 4]],
            sem_s[c % 2]).wait()

    def compute(c):
        b = c % 2
        b4 = c % 4

        def edge_body(k, carry):
            e0 = k * 16
            se_vec = s_v[b4, pl.ds(e0, 16)]
            for j in range(16):
                se = se_vec[j]
                e = e0 + j
                gu = lax.bitcast_convert_type(urows_v[b, e, :], jnp.uint32)
                lo = lax.bitcast_convert_type(gu << 16, jnp.float32)
                hi = lax.bitcast_convert_type(
                    gu & jnp.uint32(0xFFFF0000), jnp.float32)
                scaled_v[b, e, 0:16] = lo * se
                scaled_v[b, e, 16:32] = hi * se
            return carry

        lax.fori_loop(0, CHUNK // 16, edge_body, 0)

    for c in range(NCHUNK):
        if c == 0:
            for cc in range(min(2, NCHUNK)):
                for d in ld_cps(cc):
                    d.start()
            for d in ld_cps(0):
                d.wait()
            gat_cp(0).start()
        gat_cp(c).wait()
        if c >= 2:
            sca_wait(c - 2)
        if c + 2 < NCHUNK:
            for d in ld_cps(c + 2):
                d.start()
        compute(c)
        sca_start(c)
        if c + 1 < NCHUNK:
            for d in ld_cps(c + 1):
                d.wait()
            gat_cp(c + 1).start()

    if NCHUNK >= 2:
        sca_wait(NCHUNK - 2)
    sca_wait(NCHUNK - 1)

    plsc.subcore_barrier()

    @pl.when(sid == 0)
    def _():
        pltpu.sync_copy(acc_sh, out_hbm.at[cid])


_scatter = functools.partial(
    pl.kernel,
    out_type=jax.ShapeDtypeStruct((NC, N_NODES, ACC_W), jnp.float32),
    mesh=_mesh,
    compiler_params=pltpu.CompilerParams(use_tc_tiling_on_sc=False),
    scratch_types=[
        pltpu.VMEM((4, CHUNK), jnp.int32),
        pltpu.VMEM((4, CHUNK), jnp.int32),
        pltpu.VMEM((4, CHUNK), jnp.float32),
        pltpu.VMEM((2, CHUNK, C_U // 2), jnp.int32),
        pltpu.VMEM((2, CHUNK, ACC_W), jnp.float32),
        pltpu.VMEM_SHARED((N_NODES, ACC_W), jnp.float32),
        pltpu.SemaphoreType.DMA,
        pltpu.SemaphoreType.DMA,
        pltpu.SemaphoreType.DMA,
        pltpu.SemaphoreType.DMA,
        pltpu.SemaphoreType.DMA,
        pltpu.SemaphoreType.DMA,
        pltpu.SemaphoreType.DMA,
        pltpu.SemaphoreType.DMA,
    ],
)(_scatter_body)


def _final_body(p_ref, o_ref):
    a = p_ref[0, 0] + p_ref[0, 1]
    for k in range(1, NSLAB):
        a = a + p_ref[k, 0] + p_ref[k, 1]
    cnt = a[:, 32:33]
    row = lax.broadcasted_iota(jnp.int32, (N_NODES, 1), 0)
    cnt = cnt - jnp.where(row == 0, jnp.float32(NPAD), 0.0)
    cnt = cnt + (cnt == 0.0).astype(jnp.float32)
    o_ref[...] = a[:, 0:32] / cnt


def _finalize(partials):
    return pl.pallas_call(
        _final_body,
        out_shape=jax.ShapeDtypeStruct((N_NODES, C_U), jnp.float32),
    )(partials)


def kernel(u, x0, x1, ind0, ind1, W0, b0, W1, b1, W2, b2, W3, b3, W4, b4):
    s_shape = x1.shape[1:]
    ut = u.reshape(C_U, N_NODES).T
    x0t = x0.reshape(C_U, N_NODES).T
    x1t = x1.reshape(C_U, N_NODES).T
    pad = jnp.zeros((NPAD,), dtype=jnp.int32)
    ind0 = jnp.concatenate([ind0.astype(jnp.int32), pad])
    ind1 = jnp.concatenate([ind1.astype(jnp.int32), pad])

    pt, qt, utp = _precompute(ut, x0t, x1t, W0, b0.reshape(1, HID))
    bf = jnp.bfloat16
    zeros = jnp.zeros((N_NODES, ACC_W), dtype=jnp.float32)
    partials = []
    for k in range(NSLAB):
        lo, hi = k * ESLAB, (k + 1) * ESLAB
        i0k, i1k = ind0[lo:hi], ind1[lo:hi]
        g0, g1 = _gather(pt, qt, i0k, i1k)
        s = _mlp(lo, g0, g1, W1.astype(bf), b1.reshape(1, HID),
                 W2.astype(bf), b2.reshape(1, HID), W3.astype(bf),
                 b3.reshape(1, HID), W4, b4)
        partials.append(_scatter(utp, i0k, i1k, s, zeros))
    out_t = _finalize(jnp.stack(partials))
    return out_t.T.reshape((C_U,) + tuple(s_shape))


# speedup vs baseline: 1.0681x; 1.0681x over previous
import functools

import jax
import jax.numpy as jnp
from jax import lax
from jax.experimental import pallas as pl
from jax.experimental.pallas import tpu as pltpu
from jax.experimental.pallas import tpu_sc as plsc

N_NODES = 10000
N_EDGES = 320000
EPAD = 327680
NPAD = EPAD - N_EDGES
C_U = 32
HID = 128

NC = 2
NS = 16
NW = NC * NS
NSLAB = 4
ESLAB = EPAD // NSLAB
EPW = ESLAB // NW
GCHUNK = 640
NGCHUNK = EPW // GCHUNK
NJOB = 2 * NGCHUNK
CHUNK = 640
NCHUNK = EPW // CHUNK
ACC_W = 48
PW = HID // 2

_mesh = plsc.VectorSubcoreMesh(core_axis_name="c", subcore_axis_name="s")


def _pre_body(ut_ref, x0t_ref, x1t_ref, w0_ref, b0_ref, pt_ref, qt_ref,
              utp_ref):
    w0 = w0_ref[...]
    a = w0[:, 0:32]
    b = w0[:, 32:64]
    c = w0[:, 64:96]
    d = w0[:, 96:128]
    dn = (((1,), (1,)), ((), ()))
    pt = lax.dot_general(ut_ref[...], a, dn)
    pt = pt + lax.dot_general(x0t_ref[...], c, dn)
    pt = pt + lax.dot_general(x1t_ref[...], d, dn)
    pt = pt + b0_ref[...]
    qt = lax.dot_general(ut_ref[...], b, dn)

    def pack(x):
        xb = x.astype(jnp.bfloat16)
        lo = lax.convert_element_type(
            lax.bitcast_convert_type(xb[:, 0:64], jnp.uint16), jnp.uint32)
        hi = lax.convert_element_type(
            lax.bitcast_convert_type(xb[:, 64:128], jnp.uint16), jnp.uint32)
        return lax.bitcast_convert_type(lo | (hi << 16), jnp.int32)

    pt_ref[...] = pack(pt)
    qt_ref[...] = pack(qt)
    ub = ut_ref[...].astype(jnp.bfloat16)
    ulo = lax.convert_element_type(
        lax.bitcast_convert_type(ub[:, 0:16], jnp.uint16), jnp.uint32)
    uhi = lax.convert_element_type(
        lax.bitcast_convert_type(ub[:, 16:32], jnp.uint16), jnp.uint32)
    utp_ref[...] = lax.bitcast_convert_type(ulo | (uhi << 16), jnp.int32)


def _precompute(ut, x0t, x1t, w0, b0row):
    return pl.pallas_call(
        _pre_body,
        out_shape=(
            jax.ShapeDtypeStruct((N_NODES, HID // 2), jnp.int32),
            jax.ShapeDtypeStruct((N_NODES, HID // 2), jnp.int32),
            jax.ShapeDtypeStruct((N_NODES, C_U // 2), jnp.int32),
        ),
    )(ut, x0t, x1t, w0, b0row)


def _gather_body(pt_hbm, qt_hbm, i0_hbm, i1_hbm, g0_hbm, g1_hbm,
                 idx0_v, idx1_v, rows_v, si0, si1, sg0, sg1, sw0, sw1):
    wid = lax.axis_index("s") * NC + lax.axis_index("c")
    tile_base = wid * EPW
    sem_i = (si0, si1)
    sem_g = (sg0, sg1)
    sem_w = (sw0, sw1)
    isrc = (i0_hbm, i1_hbm)
    tbl = (pt_hbm, qt_hbm)
    dst = (g0_hbm, g1_hbm)
    idxb = (idx0_v, idx1_v)

    def idx_cp(j, b):
        base = tile_base + (j >> 1) * GCHUNK
        return pltpu.make_async_copy(
            isrc[b].at[pl.ds(base, GCHUNK)], idxb[b], sem_i[b])

    def gat_cp(b):
        return pltpu.make_async_copy(
            tbl[b].at[idxb[b]], rows_v.at[b], sem_g[b])

    def wb_cp(j, b):
        base = tile_base + (j >> 1) * GCHUNK
        return pltpu.make_async_copy(
            rows_v.at[b], dst[b].at[pl.ds(base, GCHUNK)], sem_w[b])

    idx_cp(0, 0).start()
    idx_cp(1, 1).start()
    idx_cp(0, 0).wait()
    gat_cp(0).start()
    gat_cp(0).wait()
    wb_cp(0, 0).start()
    idx_cp(2, 0).start()
    idx_cp(1, 1).wait()
    gat_cp(1).start()
    gat_cp(1).wait()
    wb_cp(1, 1).start()
    idx_cp(3, 1).start()
    wb_cp(0, 0).wait()
    idx_cp(2, 0).wait()
    gat_cp(0).start()

    def steady(j, b):
        gat_cp(b).wait()
        wb_cp(j, b).start()
        idx_cp(j + 2, b).start()
        wb_cp(j - 1, 1 - b).wait()
        idx_cp(j + 1, 1 - b).wait()
        gat_cp(1 - b).start()

    @pl.loop(2, NJOB - 2, step=2)
    def _(jv):
        steady(jv, 0)
        steady(jv + 1, 1)

    j = NJOB - 2
    gat_cp(0).wait()
    wb_cp(j, 0).start()
    wb_cp(j - 1, 1).wait()
    idx_cp(j + 1, 1).wait()
    gat_cp(1).start()
    gat_cp(1).wait()
    wb_cp(j + 1, 1).start()
    wb_cp(j, 0).wait()
    wb_cp(j + 1, 1).wait()


_gather = functools.partial(
    pl.kernel,
    out_type=(
        jax.ShapeDtypeStruct((ESLAB, PW), jnp.int32),
        jax.ShapeDtypeStruct((ESLAB, PW), jnp.int32),
    ),
    mesh=_mesh,
    compiler_params=pltpu.CompilerParams(use_tc_tiling_on_sc=False),
    scratch_types=[
        pltpu.VMEM((GCHUNK,), jnp.int32),
        pltpu.VMEM((GCHUNK,), jnp.int32),
        pltpu.VMEM((2, GCHUNK, PW), jnp.int32),
        pltpu.SemaphoreType.DMA,
        pltpu.SemaphoreType.DMA,
        pltpu.SemaphoreType.DMA,
        pltpu.SemaphoreType.DMA,
        pltpu.SemaphoreType.DMA,
        pltpu.SemaphoreType.DMA,
    ],
)(_gather_body)


BE = 2048
NBLK = ESLAB // BE


def _mlp_body(slab_off, g0_ref, g1_ref, w1_ref, b1_ref, w2_ref, b2_ref,
              w3_ref, b3_ref, w4_ref, b4_ref, s_ref):
    dn = (((1,), (1,)), ((), ()))
    f32 = jnp.float32

    def unpack(gref):
        gu = lax.bitcast_convert_type(gref[...], jnp.uint32)
        lo = lax.bitcast_convert_type(gu << 16, f32)
        hi = lax.bitcast_convert_type(gu & jnp.uint32(0xFFFF0000), f32)
        return jnp.concatenate([lo, hi], axis=1)

    h = unpack(g0_ref) + unpack(g1_ref)
    for w_ref, b_ref in ((w1_ref, b1_ref), (w2_ref, b2_ref), (w3_ref, b3_ref)):
        h = jax.nn.gelu(h).astype(jnp.bfloat16)
        h = lax.dot_general(h, w_ref[...], dn,
                            preferred_element_type=f32) + b_ref[...]
    h = jax.nn.gelu(h).astype(jnp.bfloat16)
    s = lax.dot_general(w4_ref[...].astype(jnp.bfloat16), h, dn,
                        preferred_element_type=f32) + b4_ref[0, 0]
    i = pl.program_id(0)
    eid = slab_off + i * BE + lax.broadcasted_iota(jnp.int32, (1, BE), 1)
    s = jnp.where(eid < N_EDGES, s, 0.0)
    s_ref[...] = s.reshape(BE)


def _mlp(slab_off, g0, g1, w1, b1r, w2, b2r, w3, b3r, w4, b4):
    wspec = pl.BlockSpec((HID, HID), lambda i: (0, 0))
    bspec = pl.BlockSpec((1, HID), lambda i: (0, 0))
    return pl.pallas_call(
        functools.partial(_mlp_body, slab_off),
        grid=(NBLK,),
        in_specs=[
            pl.BlockSpec((BE, PW), lambda i: (i, 0)),
            pl.BlockSpec((BE, PW), lambda i: (i, 0)),
            wspec, bspec, wspec, bspec, wspec, bspec,
            pl.BlockSpec((1, HID), lambda i: (0, 0)),
            pl.BlockSpec((1, 1), lambda i: (0, 0)),
        ],
        out_specs=pl.BlockSpec((BE,), lambda i: (i,)),
        out_shape=jax.ShapeDtypeStruct((ESLAB,), jnp.float32),
    )(g0, g1, w1, b1r, w2, b2r, w3, b3r, w4, b4)


def _scatter_body(utp_hbm, i0_hbm, i1_hbm, s_hbm, acc_in_hbm, out_hbm,
                  idx0_v, idx1_v, s_v, urows_v, scaled_v, acc_sh,
                  sl0, sl1, sl2, sl3, sg0, sg1, ss0, ss1):
    cid = lax.axis_index("c")
    sid = lax.axis_index("s")
    wid = sid * NC + cid
    sem_l = (sl0, sl1, sl2, sl3)
    sem_g = (sg0, sg1)
    sem_s = (ss0, ss1)

    @pl.when(sid == 0)
    def _():
        pltpu.sync_copy(acc_in_hbm.at[cid], acc_sh)

    cnt_vec = jnp.where(lax.iota(jnp.int32, 16) == 0,
                        jnp.float32(1.0), jnp.float32(0.0))

    def init_body(e, carry):
        scaled_v[0, e, 32:48] = cnt_vec
        scaled_v[1, e, 32:48] = cnt_vec
        return carry

    lax.fori_loop(0, CHUNK, init_body, 0)
    plsc.subcore_barrier()

    def ld_cps(c):
        b = c % 4
        base = wid * EPW + c * CHUNK
        return (
            pltpu.make_async_copy(
                i0_hbm.at[pl.ds(base, CHUNK)], idx0_v.at[b], sem_l[b]),
            pltpu.make_async_copy(
                i1_hbm.at[pl.ds(base, CHUNK)], idx1_v.at[b], sem_l[b]),
            pltpu.make_async_copy(
                s_hbm.at[pl.ds(base, CHUNK)], s_v.at[b], sem_l[b]),
        )

    def gat_cp(c):
        return pltpu.make_async_copy(
            utp_hbm.at[idx0_v.at[c % 4]], urows_v.at[c % 2], sem_g[c % 2])

    def sca_start(c):
        pltpu.async_copy(
            scaled_v.at[c % 2], acc_sh.at[idx1_v.at[c % 4]], sem_s[c % 2],
            add=True)

    def sca_wait(c):
        pltpu.make_async_copy(
            scaled_v.at[c % 2], acc_sh.at[idx1_v.at[c % 4]],
            sem_s[c % 2]).wait()

    def compute(c):
        b = c % 2
        b4 = c % 4

        def edge_body(k, carry):
            e0 = k * 16
            se_vec = s_v[b4, pl.ds(e0, 16)]
            for j in range(16):
                se = se_vec[j]
                e = e0 + j
                gu = lax.bitcast_convert_type(urows_v[b, e, :], jnp.uint32)
                lo = lax.bitcast_convert_type(gu << 16, jnp.float32)
                hi = lax.bitcast_convert_type(
                    gu & jnp.uint32(0xFFFF0000), jnp.float32)
                scaled_v[b, e, 0:16] = lo * se
                scaled_v[b, e, 16:32] = hi * se
            return carry

        lax.fori_loop(0, CHUNK // 16, edge_body, 0)

    for c in range(NCHUNK):
        if c == 0:
            for cc in range(min(2, NCHUNK)):
                for d in ld_cps(cc):
                    d.start()
            for d in ld_cps(0):
                d.wait()
            gat_cp(0).start()
        gat_cp(c).wait()
        if c >= 2:
            sca_wait(c - 2)
        if c + 2 < NCHUNK:
            for d in ld_cps(c + 2):
                d.start()
        compute(c)
        sca_start(c)
        if c + 1 < NCHUNK:
            for d in ld_cps(c + 1):
                d.wait()
            gat_cp(c + 1).start()

    if NCHUNK >= 2:
        sca_wait(NCHUNK - 2)
    sca_wait(NCHUNK - 1)

    plsc.subcore_barrier()

    @pl.when(sid == 0)
    def _():
        pltpu.sync_copy(acc_sh, out_hbm.at[cid])


_scatter = functools.partial(
    pl.kernel,
    out_type=jax.ShapeDtypeStruct((NC, N_NODES, ACC_W), jnp.float32),
    mesh=_mesh,
    compiler_params=pltpu.CompilerParams(use_tc_tiling_on_sc=False),
    scratch_types=[
        pltpu.VMEM((4, CHUNK), jnp.int32),
        pltpu.VMEM((4, CHUNK), jnp.int32),
        pltpu.VMEM((4, CHUNK), jnp.float32),
        pltpu.VMEM((2, CHUNK, C_U // 2), jnp.int32),
        pltpu.VMEM((2, CHUNK, ACC_W), jnp.float32),
        pltpu.VMEM_SHARED((N_NODES, ACC_W), jnp.float32),
        pltpu.SemaphoreType.DMA,
        pltpu.SemaphoreType.DMA,
        pltpu.SemaphoreType.DMA,
        pltpu.SemaphoreType.DMA,
        pltpu.SemaphoreType.DMA,
        pltpu.SemaphoreType.DMA,
        pltpu.SemaphoreType.DMA,
        pltpu.SemaphoreType.DMA,
    ],
)(_scatter_body)


def _final_body(p_ref, o_ref):
    a = p_ref[0] + p_ref[1]
    cnt = a[:, 32:33]
    row = lax.broadcasted_iota(jnp.int32, (N_NODES, 1), 0)
    cnt = cnt - jnp.where(row == 0, jnp.float32(NPAD), 0.0)
    cnt = cnt + (cnt == 0.0).astype(jnp.float32)
    o_ref[...] = a[:, 0:32] / cnt


def _finalize(partials):
    return pl.pallas_call(
        _final_body,
        out_shape=jax.ShapeDtypeStruct((N_NODES, C_U), jnp.float32),
    )(partials)


def kernel(u, x0, x1, ind0, ind1, W0, b0, W1, b1, W2, b2, W3, b3, W4, b4):
    s_shape = x1.shape[1:]
    ut = u.reshape(C_U, N_NODES).T
    x0t = x0.reshape(C_U, N_NODES).T
    x1t = x1.reshape(C_U, N_NODES).T
    pad = jnp.zeros((NPAD,), dtype=jnp.int32)
    ind0 = jnp.concatenate([ind0.astype(jnp.int32), pad])
    ind1 = jnp.concatenate([ind1.astype(jnp.int32), pad])

    pt, qt, utp = _precompute(ut, x0t, x1t, W0, b0.reshape(1, HID))
    bf = jnp.bfloat16
    acc = jnp.zeros((NC, N_NODES, ACC_W), dtype=jnp.float32)
    for k in range(NSLAB):
        lo, hi = k * ESLAB, (k + 1) * ESLAB
        i0k, i1k = ind0[lo:hi], ind1[lo:hi]
        g0, g1 = _gather(pt, qt, i0k, i1k)
        s = _mlp(lo, g0, g1, W1.astype(bf), b1.reshape(1, HID),
                 W2.astype(bf), b2.reshape(1, HID), W3.astype(bf),
                 b3.reshape(1, HID), W4, b4)
        acc = _scatter(utp, i0k, i1k, s, acc)
    out_t = _finalize(acc)
    return out_t.T.reshape((C_U,) + tuple(s_shape))
